# Initial kernel scaffold; baseline (speedup 1.0000x reference)
import jax
import jax.numpy as jnp
from jax.experimental import pallas as pl


def kernel(x, table):
    raise NotImplementedError("write your pallas kernel here")


# speedup vs baseline: 2.7994x; 2.7994x over previous
import functools

import jax
import jax.numpy as jnp
from jax import lax
from jax.experimental import pallas as pl
from jax.experimental.pallas import tpu as pltpu
from jax.experimental.pallas import tpu_sc as plsc

VOCAB = 1000
EMBED_DIM = 128
BATCH = 4096
HIST_LEN = 50

_INFO = plsc.get_sparse_core_info()
NC, NS = _INFO.num_cores, _INFO.num_subcores
NW = NC * NS
N_IDX = BATCH * HIST_LEN
PER_W = N_IDX // NW
CHUNK = 128
CHUNKS = PER_W // CHUNK


def _build_kernel():
    mesh = plsc.VectorSubcoreMesh(core_axis_name="c", subcore_axis_name="s")

    @functools.partial(
        pl.kernel,
        mesh=mesh,
        out_type=jax.ShapeDtypeStruct((NW, CHUNKS, CHUNK, EMBED_DIM),
                                      jnp.float32),
        scratch_types=[
            pltpu.VMEM((CHUNKS, CHUNK), jnp.int32),
            pltpu.VMEM((CHUNK, EMBED_DIM), jnp.float32),
            pltpu.SemaphoreType.DMA,
        ],
    )
    def gather_kernel(x_hbm, table_hbm, out_hbm, idx_v, rows_v, gsem):
        wid = lax.axis_index("s") * NC + lax.axis_index("c")
        pltpu.sync_copy(x_hbm.at[wid], idx_v)

        def step(j, carry):
            pltpu.async_copy(table_hbm.at[idx_v.at[j]], rows_v, gsem).wait()
            pltpu.sync_copy(rows_v, out_hbm.at[wid, j])
            return carry

        lax.fori_loop(0, CHUNKS, step, 0)

    return gather_kernel


_KERNEL = _build_kernel()


def kernel(x, table):
    idx = x.astype(jnp.int32).reshape(NW, CHUNKS, CHUNK)
    out = _KERNEL(idx, table)
    return out.reshape(BATCH, HIST_LEN, EMBED_DIM)


# speedup vs baseline: 2.9610x; 1.0577x over previous
import functools

import jax
import jax.numpy as jnp
from jax import lax
from jax.experimental import pallas as pl
from jax.experimental.pallas import tpu as pltpu
from jax.experimental.pallas import tpu_sc as plsc

VOCAB = 1000
EMBED_DIM = 128
BATCH = 4096
HIST_LEN = 50

_INFO = plsc.get_sparse_core_info()
NC, NS = _INFO.num_cores, _INFO.num_subcores
NW = NC * NS
N_IDX = BATCH * HIST_LEN
PER_W = N_IDX // NW
CHUNK = 128
CHUNKS = PER_W // CHUNK
NBUF = 5
NGROUP = CHUNKS // NBUF


def _build_kernel():
    mesh = plsc.VectorSubcoreMesh(core_axis_name="c", subcore_axis_name="s")

    @functools.partial(
        pl.kernel,
        mesh=mesh,
        out_type=jax.ShapeDtypeStruct((NW, CHUNKS, CHUNK, EMBED_DIM),
                                      jnp.float32),
        scratch_types=[
            pltpu.VMEM((CHUNKS, CHUNK), jnp.int32),
            pltpu.VMEM((NBUF, CHUNK, EMBED_DIM), jnp.float32),
        ]
        + [pltpu.SemaphoreType.DMA] * (2 * NBUF),
    )
    def gather_kernel(x_hbm, table_hbm, out_hbm, idx_v, rows_v, *sems):
        gsems, osems = sems[:NBUF], sems[NBUF:]
        wid = lax.axis_index("s") * NC + lax.axis_index("c")
        pltpu.sync_copy(x_hbm.at[wid], idx_v)

        def gather(j, b):
            return pltpu.make_async_copy(
                table_hbm.at[idx_v.at[j]], rows_v.at[b], gsems[b])

        def out_copy(j, b):
            return pltpu.make_async_copy(
                rows_v.at[b], out_hbm.at[wid, j], osems[b])

        for b in range(NBUF):
            gather(b, b).start()

        def body(g, carry):
            j0 = g * NBUF
            for b in range(NBUF):
                gather(j0 + b, b).wait()
                out_copy(j0 + b, b).start()
            for b in range(NBUF):
                out_copy(j0 + b, b).wait()
                gather(j0 + NBUF + b, b).start()
            return carry

        lax.fori_loop(0, NGROUP - 1, body, 0)

        jl = (NGROUP - 1) * NBUF
        for b in range(NBUF):
            gather(jl + b, b).wait()
            out_copy(jl + b, b).start()
        for b in range(NBUF):
            out_copy(jl + b, b).wait()

    return gather_kernel


_KERNEL = _build_kernel()


def kernel(x, table):
    idx = x.astype(jnp.int32).reshape(NW, CHUNKS, CHUNK)
    out = _KERNEL(idx, table)
    return out.reshape(BATCH, HIST_LEN, EMBED_DIM)


# speedup vs baseline: 4.9119x; 1.6589x over previous
import functools

import jax
import jax.numpy as jnp
from jax import lax
from jax.experimental import pallas as pl
from jax.experimental.pallas import tpu as pltpu
from jax.experimental.pallas import tpu_sc as plsc

VOCAB = 1000
EMBED_DIM = 128
BATCH = 4096
HIST_LEN = 50
HIST_PAD = 56

_INFO = plsc.get_sparse_core_info()
NC, NS = _INFO.num_cores, _INFO.num_subcores
NW = NC * NS
B_PER_W = BATCH // NW
NBUF = 8
NGROUP = B_PER_W // NBUF


def _build_kernel():
    mesh = plsc.VectorSubcoreMesh(core_axis_name="c", subcore_axis_name="s")

    @functools.partial(
        pl.kernel,
        mesh=mesh,
        out_type=jax.ShapeDtypeStruct((BATCH, HIST_LEN, EMBED_DIM),
                                      jnp.float32),
        scratch_types=[
            pltpu.VMEM((B_PER_W, HIST_PAD), jnp.int32),
            pltpu.VMEM((NBUF, HIST_LEN, EMBED_DIM), jnp.float32),
        ]
        + [pltpu.SemaphoreType.DMA] * (2 * NBUF),
    )
    def gather_kernel(x_hbm, table_hbm, out_hbm, idx_v, rows_v, *sems):
        gsems, osems = sems[:NBUF], sems[NBUF:]
        wid = lax.axis_index("s") * NC + lax.axis_index("c")
        b0 = wid * B_PER_W
        pltpu.sync_copy(x_hbm.at[wid], idx_v)

        def gather(i, b):
            return pltpu.make_async_copy(
                table_hbm.at[idx_v.at[i, pl.ds(0, HIST_LEN)]],
                rows_v.at[b], gsems[b])

        def out_copy(i, b):
            return pltpu.make_async_copy(
                rows_v.at[b], out_hbm.at[b0 + i], osems[b])

        for b in range(NBUF):
            gather(b, b).start()

        def body(g, carry):
            i0 = g * NBUF
            for b in range(NBUF):
                gather(i0 + b, b).wait()
                out_copy(i0 + b, b).start()
            for b in range(NBUF):
                out_copy(i0 + b, b).wait()
                gather(i0 + NBUF + b, b).start()
            return carry

        lax.fori_loop(0, NGROUP - 1, body, 0)

        il = (NGROUP - 1) * NBUF
        for b in range(NBUF):
            gather(il + b, b).wait()
            out_copy(il + b, b).start()
        for b in range(NBUF):
            out_copy(il + b, b).wait()

    return gather_kernel


_KERNEL = _build_kernel()


def kernel(x, table):
    idx = x.astype(jnp.int32)
    idx = jnp.pad(idx, ((0, 0), (0, HIST_PAD - HIST_LEN)))
    idx = idx.reshape(NW, B_PER_W, HIST_PAD)
    return _KERNEL(idx, table)


# speedup vs baseline: 4.9170x; 1.0010x over previous
import functools

import jax
import jax.numpy as jnp
from jax import lax
from jax.experimental import pallas as pl
from jax.experimental.pallas import tpu as pltpu
from jax.experimental.pallas import tpu_sc as plsc

VOCAB = 1000
EMBED_DIM = 128
BATCH = 4096
HIST_LEN = 50
HIST_PAD = 56

_INFO = plsc.get_sparse_core_info()
NC, NS = _INFO.num_cores, _INFO.num_subcores
NW = NC * NS
B_PER_W = BATCH // NW
NBUF = 8
NGROUP = B_PER_W // NBUF


def _build_kernel():
    mesh = plsc.VectorSubcoreMesh(core_axis_name="c", subcore_axis_name="s")

    @functools.partial(
        pl.kernel,
        mesh=mesh,
        compiler_params=pltpu.CompilerParams(use_tc_tiling_on_sc=True),
        out_type=jax.ShapeDtypeStruct((BATCH, HIST_LEN, EMBED_DIM),
                                      jnp.float32),
        scratch_types=[
            pltpu.VMEM((B_PER_W, HIST_PAD), jnp.int32),
            pltpu.VMEM((NBUF, HIST_LEN, EMBED_DIM), jnp.float32),
        ]
        + [pltpu.SemaphoreType.DMA] * (2 * NBUF),
    )
    def gather_kernel(x_hbm, table_hbm, out_hbm, idx_v, rows_v, *sems):
        gsems, osems = sems[:NBUF], sems[NBUF:]
        wid = lax.axis_index("s") * NC + lax.axis_index("c")
        b0 = wid * B_PER_W
        pltpu.sync_copy(x_hbm.at[wid], idx_v)

        def gather(i, b):
            return pltpu.make_async_copy(
                table_hbm.at[idx_v.at[i, pl.ds(0, HIST_LEN)]],
                rows_v.at[b], gsems[b])

        def out_copy(i, b):
            return pltpu.make_async_copy(
                rows_v.at[b], out_hbm.at[b0 + i], osems[b])

        for b in range(NBUF):
            gather(b, b).start()

        def body(g, carry):
            i0 = g * NBUF
            for b in range(NBUF):
                gather(i0 + b, b).wait()
                out_copy(i0 + b, b).start()
            for b in range(NBUF):
                out_copy(i0 + b, b).wait()
                gather(i0 + NBUF + b, b).start()
            return carry

        lax.fori_loop(0, NGROUP - 1, body, 0)

        il = (NGROUP - 1) * NBUF
        for b in range(NBUF):
            gather(il + b, b).wait()
            out_copy(il + b, b).start()
        for b in range(NBUF):
            out_copy(il + b, b).wait()

    return gather_kernel


_KERNEL = _build_kernel()


def kernel(x, table):
    idx = x.astype(jnp.int32)
    idx = jnp.pad(idx, ((0, 0), (0, HIST_PAD - HIST_LEN)))
    idx = idx.reshape(NW, B_PER_W, HIST_PAD)
    return _KERNEL(idx, table)


# speedup vs baseline: 7.3055x; 1.4858x over previous
import functools

import jax
import jax.numpy as jnp
from jax import lax
from jax.experimental import pallas as pl
from jax.experimental.pallas import tpu as pltpu
from jax.experimental.pallas import tpu_sc as plsc

VOCAB = 1000
EMBED_DIM = 128
BATCH = 4096
HIST_LEN = 50
HIST_PAD = 56

_INFO = plsc.get_sparse_core_info()
NC, NS = _INFO.num_cores, _INFO.num_subcores
NW = NC * NS
B_PER_W = BATCH // NW
NBUF = 8
NGROUP = B_PER_W // NBUF


def _build_kernel():
    mesh = plsc.VectorSubcoreMesh(core_axis_name="c", subcore_axis_name="s")

    @functools.partial(
        pl.kernel,
        mesh=mesh,
        out_type=jax.ShapeDtypeStruct((BATCH, HIST_LEN, EMBED_DIM),
                                      jnp.float32),
        scratch_types=[
            pltpu.VMEM((B_PER_W, HIST_PAD), jnp.int32),
            pltpu.VMEM((NBUF, HIST_LEN, EMBED_DIM), jnp.float32),
            pltpu.VMEM_SHARED((VOCAB, EMBED_DIM), jnp.float32),
        ]
        + [pltpu.SemaphoreType.DMA] * (2 * NBUF),
    )
    def gather_kernel(x_hbm, table_hbm, out_hbm, idx_v, rows_v, table_sp,
                      *sems):
        gsems, osems = sems[:NBUF], sems[NBUF:]
        sid = lax.axis_index("s")
        wid = sid * NC + lax.axis_index("c")
        b0 = wid * B_PER_W

        @pl.when(sid == 0)
        def _stage_table():
            pltpu.sync_copy(table_hbm, table_sp)

        pltpu.sync_copy(x_hbm.at[wid], idx_v)
        plsc.subcore_barrier()

        def gather(i, b):
            return pltpu.make_async_copy(
                table_sp.at[idx_v.at[i, pl.ds(0, HIST_LEN)]],
                rows_v.at[b], gsems[b])

        def out_copy(i, b):
            return pltpu.make_async_copy(
                rows_v.at[b], out_hbm.at[b0 + i], osems[b])

        for b in range(NBUF):
            gather(b, b).start()

        def body(g, carry):
            i0 = g * NBUF
            for b in range(NBUF):
                gather(i0 + b, b).wait()
                out_copy(i0 + b, b).start()
            for b in range(NBUF):
                out_copy(i0 + b, b).wait()
                gather(i0 + NBUF + b, b).start()
            return carry

        lax.fori_loop(0, NGROUP - 1, body, 0)

        il = (NGROUP - 1) * NBUF
        for b in range(NBUF):
            gather(il + b, b).wait()
            out_copy(il + b, b).start()
        for b in range(NBUF):
            out_copy(il + b, b).wait()

    return gather_kernel


_KERNEL = _build_kernel()


def kernel(x, table):
    idx = x.astype(jnp.int32)
    idx = jnp.pad(idx, ((0, 0), (0, HIST_PAD - HIST_LEN)))
    idx = idx.reshape(NW, B_PER_W, HIST_PAD)
    return _KERNEL(idx, table)
